# Initial kernel scaffold; baseline (speedup 1.0000x reference)
import jax
import jax.numpy as jnp
from jax.experimental import pallas as pl


def kernel(hidden_states, W):
    raise NotImplementedError("write your pallas kernel here")


# speedup vs baseline: 1.7339x; 1.7339x over previous
import functools

import jax
import jax.numpy as jnp
from jax import lax
from jax.experimental import pallas as pl
from jax.experimental.pallas import tpu as pltpu
from jax.experimental.pallas import tpu_sc as plsc

B, S, H = 4, 8192, 768
E = 64
K = 8
N = B * S
NW = 32
TPW = N // NW
BT = 1024
NB = N // BT
AUX_COEF = 0.01
Z_COEF = 0.01
L = 16


def _tc_body(x_ref, wt_ref, pt_ref, stats_ref):
    b = pl.program_id(0)
    x = x_ref[...]
    lg = jnp.dot(x, wt_ref[...], preferred_element_type=jnp.float32)
    lg = jnp.clip(lg, -10.0, 10.0)
    m = jnp.max(lg, axis=1, keepdims=True)
    ex = jnp.exp(lg - m)
    s = jnp.sum(ex, axis=1, keepdims=True)
    probs = ex / s
    pt_ref[0] = probs.T
    logz = m + jnp.log(s)

    @pl.when(b == 0)
    def _():
        stats_ref[...] = jnp.zeros_like(stats_ref)

    stats_ref[0:1, 0:E] += jnp.sum(probs, axis=0, keepdims=True)
    stats_ref[1:2, 0:1] += jnp.sum(logz * logz, axis=0, keepdims=True)


_tc_call = pl.pallas_call(
    _tc_body,
    grid=(NB,),
    in_specs=[
        pl.BlockSpec((BT, H), lambda b: (b, 0)),
        pl.BlockSpec((H, E), lambda b: (0, 0)),
    ],
    out_specs=[
        pl.BlockSpec((1, E, BT), lambda b: (b, 0, 0)),
        pl.BlockSpec((8, 128), lambda b: (0, 0)),
    ],
    out_shape=[
        jax.ShapeDtypeStruct((NW, E, TPW), jnp.float32),
        jax.ShapeDtypeStruct((8, 128), jnp.float32),
    ],
)


def _tc2_body(pt_ref, thr_ref, cnt_ref):
    b = pl.program_id(0)
    probs = pt_ref[0]
    tv = thr_ref[0, 0:1]
    tid = thr_ref[0, 1:2]
    eid = lax.broadcasted_iota(jnp.int32, (E, TPW), 0).astype(jnp.float32)
    sel = ((probs > tv) | ((probs == tv) & (eid <= tid))).astype(jnp.float32)

    @pl.when(b == 0)
    def _():
        cnt_ref[...] = jnp.zeros_like(cnt_ref)

    cnt_ref[:, 0:1] += jnp.sum(sel, axis=1, keepdims=True)


_tc2_call = pl.pallas_call(
    _tc2_body,
    grid=(NB,),
    in_specs=[
        pl.BlockSpec((1, E, TPW), lambda b: (b, 0, 0)),
        pl.BlockSpec((1, 2, TPW), lambda b: (b, 0, 0)),
    ],
    out_specs=pl.BlockSpec((E, 128), lambda b: (0, 0)),
    out_shape=jax.ShapeDtypeStruct((E, 128), jnp.float32),
)


_NET8 = [(0, 1), (2, 3), (4, 5), (6, 7), (0, 2), (1, 3), (4, 6), (5, 7),
         (1, 2), (5, 6), (0, 4), (3, 7), (1, 5), (2, 6), (1, 4), (3, 6),
         (2, 4), (3, 5), (3, 4)]


def _ce(a, b):
    av, ai = a
    bv, bi = b
    p = av > bv
    return (
        (jnp.maximum(av, bv), jnp.where(p, ai, bi)),
        (jnp.minimum(av, bv), jnp.where(p, bi, ai)),
    )


def _sort8(g):
    g = list(g)
    for i, j in _NET8:
        g[i], g[j] = _ce(g[i], g[j])
    return g


def _merge8(a, b):
    c = [_ce(a[i], b[7 - i])[0] for i in range(8)]
    for step in (4, 2, 1):
        nc = list(c)
        for i in range(8):
            j = i ^ step
            if i < j:
                nc[i], nc[j] = _ce(c[i], c[j])
        c = nc
    return c


_sc_mesh = plsc.VectorSubcoreMesh(
    core_axis_name="c", subcore_axis_name="s", num_cores=2, num_subcores=16
)


@functools.partial(
    pl.kernel,
    out_type=[
        jax.ShapeDtypeStruct((NW, K, TPW), jnp.float32),
        jax.ShapeDtypeStruct((NW, K, TPW), jnp.int32),
        jax.ShapeDtypeStruct((NW, 2, TPW), jnp.float32),
    ],
    mesh=_sc_mesh,
    scratch_types=[
        pltpu.VMEM((E, TPW), jnp.float32),
        pltpu.VMEM((K, TPW), jnp.float32),
        pltpu.VMEM((K, TPW), jnp.int32),
        pltpu.VMEM((2, TPW), jnp.float32),
    ],
)
def _sc_topk(pt_hbm, w_hbm, i_hbm, t_hbm, in_v, w_v, i_v, thr_v):
    wid = lax.axis_index("s") * 2 + lax.axis_index("c")
    pltpu.sync_copy(pt_hbm.at[wid], in_v)

    def group(g, _):
        t0 = g * L

        def sorted_group(g8):
            e0 = g8 * 8
            return _sort8([
                (in_v[e0 + i, pl.ds(t0, L)],
                 jnp.full((L,), float(e0 + i), jnp.float32))
                for i in range(8)
            ])

        m01 = _merge8(sorted_group(0), sorted_group(1))
        m23 = _merge8(sorted_group(2), sorted_group(3))
        m03 = _merge8(m01, m23)
        m45 = _merge8(sorted_group(4), sorted_group(5))
        m67 = _merge8(sorted_group(6), sorted_group(7))
        m47 = _merge8(m45, m67)
        top = _merge8(m03, m47)

        ssum = top[0][0]
        for j in range(1, K):
            ssum = ssum + top[j][0]
        inv = 1.0 / ssum
        for j in range(K):
            w_v[j, pl.ds(t0, L)] = top[j][0] * inv
            i_v[j, pl.ds(t0, L)] = top[j][1].astype(jnp.int32)
        thr_v[0, pl.ds(t0, L)] = top[K - 1][0]
        thr_v[1, pl.ds(t0, L)] = top[K - 1][1]
        return 0

    lax.fori_loop(0, TPW // L, group, 0)

    pltpu.sync_copy(w_v, w_hbm.at[wid])
    pltpu.sync_copy(i_v, i_hbm.at[wid])
    pltpu.sync_copy(thr_v, t_hbm.at[wid])


def kernel(hidden_states, W):
    x = hidden_states.reshape(N, H)
    pt, stats = _tc_call(x, W.T)
    w_t, i_t, thr = _sc_topk(pt)
    cnts = _tc2_call(pt, thr)
    router_weights = w_t.transpose(0, 2, 1).reshape(B, S, K)
    selected_experts = i_t.transpose(0, 2, 1).reshape(B, S, K)
    tokens_per_expert = cnts[:, 0] / N
    router_prob_per_expert = stats[0, :E] / N
    load_balancing_loss = (
        E * jnp.sum(tokens_per_expert * router_prob_per_expert) * AUX_COEF
    )
    router_z_loss = stats[1, 0] / N * Z_COEF
    return router_weights, selected_experts, load_balancing_loss, router_z_loss
